# baseline (speedup 1.0000x reference)
import functools

import jax
import jax.numpy as jnp
from jax import lax
from jax.experimental import pallas as pl
from jax.experimental.pallas import tpu as pltpu
from jax.experimental.pallas import tpu_sc as plsc

B = 4096
F = 26
K = 32
N = B * F
NC = 2
NS = 16
NW = NC * NS
BT = 128
PER_W = F * BT
KT = K // 8


def _fm_sc(idx_hbm, val_hbm, table_hbm, out_hbm,
           idx_v, val_v, rows_v, obuf_v,
           gsem0, gsem1, osem0, osem1):
    wid = lax.axis_index("s") * NC + lax.axis_index("c")
    base = wid * PER_W
    pltpu.sync_copy(idx_hbm.at[pl.ds(base, PER_W)], idx_v)
    pltpu.sync_copy(val_hbm.at[pl.ds(base, PER_W)], val_v)

    gsems = (gsem0, gsem1)
    osems = (osem0, osem1)
    iota = lax.iota(jnp.int32, 16)

    def fire_gather(f):
        p = f & 1
        return pltpu.async_copy(
            table_hbm.at[idx_v.at[pl.ds(f * BT, BT)]],
            rows_v.at[p], gsems[p])

    def compute(f):
        p = f & 1
        vb = f * BT

        def bloop(g, _):
            g16 = g * 16
            rowi = g16 + iota
            vals = val_v[pl.ds(vb + g16, 16)]

            def kloop(k, _):
                col = jnp.full((16,), k, jnp.int32)
                v = plsc.load_gather(rows_v.at[p], [rowi, col])
                obuf_v[p, pl.ds(k * BT + g16, 16)] = v * vals
                return _

            lax.fori_loop(0, K, kloop, 0, unroll=8)
            return _

        lax.fori_loop(0, 8, bloop, 0)

    def fire_out(f):
        p = f & 1
        cps = []
        for kt in range(KT):
            cps.append(pltpu.async_copy(
                obuf_v.at[p, pl.ds(kt * 1024, 1024)],
                out_hbm.at[f, kt, wid], osems[p]))
        return cps

    pending_out = [None, None]
    gathers = [None, None]
    gathers[0] = fire_gather(0)
    for f in range(F):
        p = f & 1
        if f + 1 < F:
            gathers[1 - p] = fire_gather(f + 1)
        gathers[p].wait()
        if pending_out[p] is not None:
            for c in pending_out[p]:
                c.wait()
        compute(f)
        pending_out[p] = fire_out(f)
    for pend in pending_out:
        if pend is not None:
            for c in pend:
                c.wait()


@jax.jit
def _fm(idx_flat, val_flat, table):
    mesh = plsc.VectorSubcoreMesh(core_axis_name="c", subcore_axis_name="s")
    run = functools.partial(
        pl.kernel,
        mesh=mesh,
        out_type=jax.ShapeDtypeStruct((F, KT, NW, 1024), jnp.float32),
        scratch_types=[
            pltpu.VMEM((PER_W,), jnp.int32),
            pltpu.VMEM((PER_W,), jnp.float32),
            pltpu.VMEM((2, BT, K), jnp.float32),
            pltpu.VMEM((2, K * BT), jnp.float32),
            pltpu.SemaphoreType.DMA,
            pltpu.SemaphoreType.DMA,
            pltpu.SemaphoreType.DMA,
            pltpu.SemaphoreType.DMA,
        ],
        compiler_params=pltpu.CompilerParams(
            use_tc_tiling_on_sc=False, needs_layout_passes=False),
    )(_fm_sc)
    return run(idx_flat, val_flat, table)


def kernel(nonzero_index, nonzero_value, table):
    def to_blocks(x):
        return (x.reshape(NW, BT, F).transpose(0, 2, 1).reshape(N))

    idx_flat = to_blocks(nonzero_index.astype(jnp.int32))
    val_flat = to_blocks(nonzero_value)
    o5 = _fm(idx_flat, val_flat, table)
    o = o5.reshape(F, KT, NW, 8, BT).transpose(2, 4, 0, 1, 3)
    return o.reshape(B, F, K)
